# Initial kernel scaffold; baseline (speedup 1.0000x reference)
import jax
import jax.numpy as jnp
from jax.experimental import pallas as pl


def kernel(user_ids, item_ids, sparse_features, user_table, item_table, sparse_tables, wide_W, wide_b, dW0, db0, dW1, db1, dW2, db2, dW3, db3, tW, tb):
    raise NotImplementedError("write your pallas kernel here")


# speedup vs baseline: 1.4313x; 1.4313x over previous
import functools

import jax
import jax.numpy as jnp
from jax.experimental import pallas as pl
from jax.experimental.pallas import tpu as pltpu
from jax.experimental.pallas import tpu_sc as plsc

_B = 16384
_D = 32
_F = 26
_SPARSE_V = 100000
_DIN = (_F + 2) * _D
_H = 2 * _D
_W = 128
_BB = 512


def _gather_pipeline(table_hbm, idx_hbm, out_hbm, n):
    def body(i_vmem, o_vmem):
        pltpu.sync_copy(table_hbm.at[i_vmem.at[0]], o_vmem)

    pltpu.emit_pipeline(
        body,
        grid=(n // _W,),
        in_specs=[pl.BlockSpec((1, _W), lambda i: (0, i))],
        out_specs=[pl.BlockSpec((_W, _D), lambda i: (i, 0))],
        core_axis_name=("c", "s"),
        dimension_semantics=(pltpu.PARALLEL,),
    )(idx_hbm, out_hbm)


@functools.cache
def _sc_gather_kernel():
    mesh = plsc.VectorSubcoreMesh(core_axis_name="c", subcore_axis_name="s")

    @functools.partial(
        pl.kernel,
        out_type=(
            jax.ShapeDtypeStruct((_B, _D), jnp.float32),
            jax.ShapeDtypeStruct((_B, _D), jnp.float32),
            jax.ShapeDtypeStruct((_B * _F, _D), jnp.float32),
        ),
        mesh=mesh,
        compiler_params=pltpu.CompilerParams(use_tc_tiling_on_sc=False),
    )
    def sc_gather(ut_hbm, it_hbm, st_hbm, ui_hbm, ii_hbm, si_hbm,
                  uo_hbm, io_hbm, so_hbm):
        _gather_pipeline(ut_hbm, ui_hbm, uo_hbm, _B)
        _gather_pipeline(it_hbm, ii_hbm, io_hbm, _B)
        _gather_pipeline(st_hbm, si_hbm, so_hbm, _B * _F)

    return sc_gather


def _dense_body(xu_ref, xi_ref, xs_ref, wW_ref, wb_ref, w0_ref, b0_ref,
                w1_ref, b1_ref, w2_ref, b2_ref, w3_ref, b3_ref,
                tw_ref, tb_ref, o_ref):
    dot = lambda a, b: jax.lax.dot_general(
        a, b, (((1,), (0,)), ((), ())), preferred_element_type=jnp.float32)
    xu = xu_ref[...]
    xi = xi_ref[...]
    xs = xs_ref[...]
    w0 = w0_ref[...]
    h = dot(xu, w0[0:_D]) + dot(xi, w0[_D:2 * _D]) + dot(xs, w0[2 * _D:])
    h = jax.nn.relu(h + b0_ref[...])
    h = jax.nn.relu(dot(h, w1_ref[...]) + b1_ref[...])
    h = jax.nn.relu(dot(h, w2_ref[...]) + b2_ref[...])
    deep = dot(h, w3_ref[...]) + b3_ref[...]
    wW = wW_ref[...]
    wide = (dot(xu, wW[0:_D]) + dot(xi, wW[_D:2 * _D]) + dot(xs, wW[2 * _D:])
            + wb_ref[...])
    tw = tw_ref[...]
    logit = (jnp.sum(wide * tw[:, 0:_D], axis=1, keepdims=True)
             + jnp.sum(deep * tw[:, _D:], axis=1, keepdims=True)
             + tb_ref[...])
    o_ref[...] = jax.nn.sigmoid(logit)


def _dense_forward(xu, xi, xs, wide_W, wide_b, dW0, db0, dW1, db1,
                   dW2, db2, dW3, db3, tW, tb):
    row = lambda i: (i, 0)
    full = lambda i: (0, 0)
    return pl.pallas_call(
        _dense_body,
        grid=(_B // _BB,),
        in_specs=[
            pl.BlockSpec((_BB, _D), row),
            pl.BlockSpec((_BB, _D), row),
            pl.BlockSpec((_BB, _F * _D), row),
            pl.BlockSpec((_DIN, _D), full),
            pl.BlockSpec((1, _D), full),
            pl.BlockSpec((_DIN, _H), full),
            pl.BlockSpec((1, _H), full),
            pl.BlockSpec((_H, _H), full),
            pl.BlockSpec((1, _H), full),
            pl.BlockSpec((_H, _H), full),
            pl.BlockSpec((1, _H), full),
            pl.BlockSpec((_H, _D), full),
            pl.BlockSpec((1, _D), full),
            pl.BlockSpec((1, 2 * _D), full),
            pl.BlockSpec((1, 1), full),
        ],
        out_specs=pl.BlockSpec((_BB, 1), row),
        out_shape=jax.ShapeDtypeStruct((_B, 1), jnp.float32),
    )(xu, xi, xs, wide_W, wide_b.reshape(1, _D), dW0, db0.reshape(1, _H),
      dW1, db1.reshape(1, _H), dW2, db2.reshape(1, _H), dW3,
      db3.reshape(1, _D), tW.reshape(1, 2 * _D), tb.reshape(1, 1))


def kernel(user_ids, item_ids, sparse_features, user_table, item_table,
           sparse_tables, wide_W, wide_b, dW0, db0, dW1, db1, dW2, db2,
           dW3, db3, tW, tb):
    offs = jnp.arange(_F, dtype=jnp.int32) * _SPARSE_V
    sp_idx = (sparse_features.astype(jnp.int32) + offs[None, :]).reshape(1, _B * _F)
    u_idx = user_ids.astype(jnp.int32).reshape(1, _B)
    i_idx = item_ids.astype(jnp.int32).reshape(1, _B)
    st_flat = sparse_tables.reshape(_F * _SPARSE_V, _D)
    u_emb, i_emb, s_rows = _sc_gather_kernel()(user_table, item_table, st_flat,
                                               u_idx, i_idx, sp_idx)
    xs = s_rows.reshape(_B, _F * _D)
    return _dense_forward(u_emb, i_emb, xs, wide_W, wide_b, dW0, db0,
                          dW1, db1, dW2, db2, dW3, db3, tW, tb)
